# baseline (speedup 1.0000x reference)
import functools

import jax
import jax.numpy as jnp
from jax import lax
from jax.experimental import pallas as pl
from jax.experimental.pallas import tpu as pltpu
from jax.experimental.pallas import tpu_sc as plsc

N_ATOMS = 10000
N_BONDS = 160000
MAX_NB = 16
H = 256
DEPTH = 3

NC, NS, L = 2, 16, 16
NW = NC * NS
NCOL = H // L


@functools.cache
def _sc_mesh():
    return plsc.VectorSubcoreMesh(core_axis_name="c", subcore_axis_name="s")


CA = 8
G_ROWS = CA * MAX_NB
A_CHUNKS = 40
APW = CA * A_CHUNKS
A_PAD = APW * NW


def _gather_sum_body(msg_hbm, a2b_hbm, out_hbm, idx_v,
                     rows0, rows1, out0, out1, sem0, sem1, osem0, osem1):
    wid = lax.axis_index("s") * NC + lax.axis_index("c")
    abase = wid * APW
    pltpu.sync_copy(a2b_hbm.at[pl.ds(abase * MAX_NB, APW * MAX_NB)], idx_v)

    def gsrc(c):
        return msg_hbm.at[idx_v.at[pl.ds(c * G_ROWS, G_ROWS)]]

    pltpu.async_copy(gsrc(0), rows0, sem0)

    def body(c2, carry):
        i = 2 * c2
        pltpu.async_copy(gsrc(i + 1), rows1, sem1)
        pltpu.make_async_copy(gsrc(i), rows0, sem0).wait()

        @pl.when(c2 > 0)
        def _():
            pltpu.make_async_copy(out0, out_hbm.at[pl.ds(abase, CA)],
                                  osem0).wait()

        def atom0(a, cc):
            for j in range(NCOL):
                acc = rows0[a * MAX_NB, pl.ds(j * L, L)]
                for k in range(1, MAX_NB):
                    acc = acc + rows0[a * MAX_NB + k, pl.ds(j * L, L)]
                out0[a, pl.ds(j * L, L)] = acc
            return cc

        lax.fori_loop(0, CA, atom0, 0)
        pltpu.async_copy(out0, out_hbm.at[pl.ds(abase + i * CA, CA)], osem0)

        nxt = jnp.minimum(i + 2, A_CHUNKS - 1)
        pltpu.async_copy(gsrc(nxt), rows0, sem0)
        pltpu.make_async_copy(gsrc(i + 1), rows1, sem1).wait()

        @pl.when(c2 > 0)
        def _():
            pltpu.make_async_copy(out1, out_hbm.at[pl.ds(abase, CA)],
                                  osem1).wait()

        def atom1(a, cc):
            for j in range(NCOL):
                acc = rows1[a * MAX_NB, pl.ds(j * L, L)]
                for k in range(1, MAX_NB):
                    acc = acc + rows1[a * MAX_NB + k, pl.ds(j * L, L)]
                out1[a, pl.ds(j * L, L)] = acc
            return cc

        lax.fori_loop(0, CA, atom1, 0)
        pltpu.async_copy(out1, out_hbm.at[pl.ds(abase + (i + 1) * CA, CA)],
                         osem1)
        return carry

    lax.fori_loop(0, A_CHUNKS // 2, body, 0)
    pltpu.make_async_copy(gsrc(A_CHUNKS - 1), rows0, sem0).wait()
    pltpu.make_async_copy(out0, out_hbm.at[pl.ds(abase, CA)], osem0).wait()
    pltpu.make_async_copy(out1, out_hbm.at[pl.ds(abase, CA)], osem1).wait()


@functools.cache
def _gather_sum():
    return pl.kernel(
        _gather_sum_body,
        out_type=jax.ShapeDtypeStruct((A_PAD, H), jnp.float32),
        mesh=_sc_mesh(),
        scratch_types=[
            pltpu.VMEM((APW * MAX_NB,), jnp.int32),
            pltpu.VMEM((G_ROWS, H), jnp.float32),
            pltpu.VMEM((G_ROWS, H), jnp.float32),
            pltpu.VMEM((CA, H), jnp.float32),
            pltpu.VMEM((CA, H), jnp.float32),
            pltpu.SemaphoreType.DMA,
            pltpu.SemaphoreType.DMA,
            pltpu.SemaphoreType.DMA,
            pltpu.SemaphoreType.DMA,
        ],
    )


CB = 40
B_CHUNKS = 125
BPW = CB * B_CHUNKS


def _edge_update_body(amsg_hbm, msg_hbm, b2a_hbm, b2revb_hbm, out_hbm,
                      ia_v, ir_v, ra0, ra1, rr0, rr1, mv0, mv1,
                      sa0, sa1, sr0, sr1, os0, os1):
    wid = lax.axis_index("s") * NC + lax.axis_index("c")
    bbase = wid * BPW
    pltpu.sync_copy(b2a_hbm.at[pl.ds(bbase, BPW)], ia_v)
    pltpu.sync_copy(b2revb_hbm.at[pl.ds(bbase, BPW)], ir_v)

    def asrc(c):
        return amsg_hbm.at[ia_v.at[pl.ds(c * CB, CB)]]

    def rsrc(c):
        return msg_hbm.at[ir_v.at[pl.ds(c * CB, CB)]]

    def start(c, ra, rr, sa, sr):
        pltpu.async_copy(asrc(c), ra, sa)
        pltpu.async_copy(rsrc(c), rr, sr)

    def sub(c, ra, rr, mv, osem):
        def row(r, cc):
            for j in range(NCOL):
                mv[r, pl.ds(j * L, L)] = (ra[r, pl.ds(j * L, L)]
                                          - rr[r, pl.ds(j * L, L)])
            return cc

        lax.fori_loop(0, CB, row, 0)
        pltpu.async_copy(mv, out_hbm.at[pl.ds(bbase + c * CB, CB)], osem)

    start(0, ra0, rr0, sa0, sr0)

    def body(c2, carry):
        i = 2 * c2
        start(i + 1, ra1, rr1, sa1, sr1)
        pltpu.make_async_copy(asrc(i), ra0, sa0).wait()
        pltpu.make_async_copy(rsrc(i), rr0, sr0).wait()

        @pl.when(c2 > 0)
        def _():
            pltpu.make_async_copy(mv0, out_hbm.at[pl.ds(bbase, CB)],
                                  os0).wait()

        sub(i, ra0, rr0, mv0, os0)
        start(i + 2, ra0, rr0, sa0, sr0)
        pltpu.make_async_copy(asrc(i + 1), ra1, sa1).wait()
        pltpu.make_async_copy(rsrc(i + 1), rr1, sr1).wait()

        @pl.when(c2 > 0)
        def _():
            pltpu.make_async_copy(mv1, out_hbm.at[pl.ds(bbase, CB)],
                                  os1).wait()

        sub(i + 1, ra1, rr1, mv1, os1)
        return carry

    lax.fori_loop(0, (B_CHUNKS - 1) // 2, body, 0)
    pltpu.make_async_copy(asrc(B_CHUNKS - 1), ra0, sa0).wait()
    pltpu.make_async_copy(rsrc(B_CHUNKS - 1), rr0, sr0).wait()
    pltpu.make_async_copy(mv0, out_hbm.at[pl.ds(bbase, CB)], os0).wait()
    sub(B_CHUNKS - 1, ra0, rr0, mv0, os0)
    pltpu.make_async_copy(mv0, out_hbm.at[pl.ds(bbase, CB)], os0).wait()
    pltpu.make_async_copy(mv1, out_hbm.at[pl.ds(bbase, CB)], os1).wait()


@functools.cache
def _edge_update():
    return pl.kernel(
        _edge_update_body,
        out_type=jax.ShapeDtypeStruct((N_BONDS, H), jnp.float32),
        mesh=_sc_mesh(),
        scratch_types=[
            pltpu.VMEM((BPW,), jnp.int32),
            pltpu.VMEM((BPW,), jnp.int32),
            pltpu.VMEM((CB, H), jnp.float32),
            pltpu.VMEM((CB, H), jnp.float32),
            pltpu.VMEM((CB, H), jnp.float32),
            pltpu.VMEM((CB, H), jnp.float32),
            pltpu.VMEM((CB, H), jnp.float32),
            pltpu.VMEM((CB, H), jnp.float32),
            pltpu.SemaphoreType.DMA,
            pltpu.SemaphoreType.DMA,
            pltpu.SemaphoreType.DMA,
            pltpu.SemaphoreType.DMA,
            pltpu.SemaphoreType.DMA,
            pltpu.SemaphoreType.DMA,
        ],
    )


BB_A = 1600


def _bf(x):
    return x.astype(jnp.bfloat16)


def _proj_inp_body(fb_ref, wi_ref, inp_ref):
    inp_ref[...] = jnp.dot(_bf(fb_ref[...]), wi_ref[...],
                           preferred_element_type=jnp.float32)


def _proj_inp(f_bonds, w_i_t):
    fdim = f_bonds.shape[1]
    return pl.pallas_call(
        _proj_inp_body,
        grid=(N_BONDS // BB_A,),
        in_specs=[
            pl.BlockSpec((BB_A, fdim), lambda i: (i, 0)),
            pl.BlockSpec((fdim, H), lambda i: (0, 0)),
        ],
        out_specs=pl.BlockSpec((BB_A, H), lambda i: (i, 0)),
        out_shape=jax.ShapeDtypeStruct((N_BONDS, H), jnp.float32),
    )(f_bonds, w_i_t)


BB_D = 1600


NB_D = N_BONDS // BB_D


def _gru_body(inp_ref, m_ref, mprev_ref, wih_ref, whh_ref, bih_ref, bhh_ref,
              out_ref, gi_s, gh_s):
    i = pl.program_id(0)
    s = i % 2
    sp = 1 - s
    gi_s[s] = jnp.dot(_bf(inp_ref[...]), wih_ref[...],
                      preferred_element_type=jnp.float32)
    gh_s[s] = jnp.dot(_bf(m_ref[...]), whh_ref[...],
                      preferred_element_type=jnp.float32)
    gi = gi_s[sp] + bih_ref[...]
    gh = gh_s[sp] + bhh_ref[...]
    m = mprev_ref[...]
    r = jax.nn.sigmoid(gi[:, :H] + gh[:, :H])
    z = jax.nn.sigmoid(gi[:, H:2 * H] + gh[:, H:2 * H])
    n = jnp.tanh(gi[:, 2 * H:] + r * gh[:, 2 * H:])
    out_ref[...] = (1.0 - z) * n + z * m

    @pl.when(i == 1)
    def _():
        out_ref[0:1, :] = jnp.zeros((1, H), jnp.float32)


def _gru(inp, m, w_ih_t, w_hh_t, b_ih_row, b_hh_row):
    last = NB_D - 1
    return pl.pallas_call(
        _gru_body,
        grid=(NB_D + 1,),
        in_specs=[
            pl.BlockSpec((BB_D, H), lambda i: (jnp.minimum(i, last), 0)),
            pl.BlockSpec((BB_D, H), lambda i: (jnp.minimum(i, last), 0)),
            pl.BlockSpec((BB_D, H), lambda i: (jnp.maximum(i - 1, 0), 0)),
            pl.BlockSpec((H, 3 * H), lambda i: (0, 0)),
            pl.BlockSpec((H, 3 * H), lambda i: (0, 0)),
            pl.BlockSpec((1, 3 * H), lambda i: (0, 0)),
            pl.BlockSpec((1, 3 * H), lambda i: (0, 0)),
        ],
        out_specs=pl.BlockSpec((BB_D, H), lambda i: (jnp.maximum(i - 1, 0), 0)),
        out_shape=jax.ShapeDtypeStruct((N_BONDS, H), jnp.float32),
        scratch_shapes=[
            pltpu.VMEM((2, BB_D, 3 * H), jnp.float32),
            pltpu.VMEM((2, BB_D, 3 * H), jnp.float32),
        ],
    )(inp, m, m, w_ih_t, w_hh_t, b_ih_row, b_hh_row)


BA_E = 2000


def _out_body(fa_ref, am_ref, w1_ref, w2_ref, b_ref, mask_ref, out_ref):
    acc = jnp.dot(_bf(fa_ref[...]), w1_ref[...],
                  preferred_element_type=jnp.float32)
    acc = acc + jnp.dot(_bf(am_ref[...]), w2_ref[...],
                        preferred_element_type=jnp.float32)
    acc = jnp.maximum(acc + b_ref[...], 0.0)
    out_ref[...] = acc * mask_ref[...]


def _out_proj(f_atoms, a_msg_pad, w1_t, w2_t, b_row, mask):
    return pl.pallas_call(
        _out_body,
        grid=(N_ATOMS // BA_E,),
        in_specs=[
            pl.BlockSpec((BA_E, f_atoms.shape[1]), lambda i: (i, 0)),
            pl.BlockSpec((BA_E, H), lambda i: (i, 0)),
            pl.BlockSpec((f_atoms.shape[1], H), lambda i: (0, 0)),
            pl.BlockSpec((H, H), lambda i: (0, 0)),
            pl.BlockSpec((1, H), lambda i: (0, 0)),
            pl.BlockSpec((BA_E, 1), lambda i: (i, 0)),
        ],
        out_specs=pl.BlockSpec((BA_E, H), lambda i: (i, 0)),
        out_shape=jax.ShapeDtypeStruct((N_ATOMS, H), jnp.float32),
    )(f_atoms, a_msg_pad, w1_t, w2_t, b_row, mask)


def kernel(f_atoms, f_bonds, a2b, b2a, b2revb, undirected_b2a, mask,
           W_i, W_ih, W_hh, b_ih, b_hh, W_o_w, W_o_b):
    del undirected_b2a
    afdim = f_atoms.shape[1]
    w_i_t = _bf(W_i.T)
    w_ih_t = _bf(W_ih.T)
    w_hh_t = _bf(W_hh.T)
    w1_t = _bf(W_o_w[:, :afdim].T)
    w2_t = _bf(W_o_w[:, afdim:].T)

    n_pad = A_PAD * MAX_NB - N_ATOMS * MAX_NB
    a2b_flat = jnp.concatenate([
        a2b.reshape(-1).astype(jnp.int32),
        jnp.arange(n_pad, dtype=jnp.int32),
    ])
    b2a = b2a.astype(jnp.int32)
    b2revb = b2revb.astype(jnp.int32)

    inp = _proj_inp(f_bonds, w_i_t)

    msg = inp
    for _ in range(DEPTH - 1):
        amsg = _gather_sum()(msg, a2b_flat)
        m = _edge_update()(amsg, msg, b2a, b2revb)
        msg = _gru(inp, m, w_ih_t, w_hh_t,
                   b_ih.reshape(1, -1), b_hh.reshape(1, -1))

    amsg = _gather_sum()(msg, a2b_flat)
    return _out_proj(f_atoms, amsg, w1_t, w2_t, W_o_b.reshape(1, -1), mask)


# speedup vs baseline: 1.2039x; 1.2039x over previous
import functools

import jax
import jax.numpy as jnp
from jax import lax
from jax.experimental import pallas as pl
from jax.experimental.pallas import tpu as pltpu
from jax.experimental.pallas import tpu_sc as plsc

N_ATOMS = 10000
N_BONDS = 160000
MAX_NB = 16
H = 256
DEPTH = 3

NC, NS, L = 2, 16, 16
NW = NC * NS
NCOL = H // L


@functools.cache
def _sc_mesh():
    return plsc.VectorSubcoreMesh(core_axis_name="c", subcore_axis_name="s")


CA = 8
G_ROWS = CA * MAX_NB
A_CHUNKS = 40
APW = CA * A_CHUNKS
A_PAD = APW * NW


def _gather_sum_body(msg_hbm, a2b_hbm, out_hbm, idx_v,
                     rows0, rows1, out0, out1, sem0, sem1, osem0, osem1):
    wid = lax.axis_index("s") * NC + lax.axis_index("c")
    abase = wid * APW
    pltpu.sync_copy(a2b_hbm.at[pl.ds(abase * MAX_NB, APW * MAX_NB)], idx_v)

    def gsrc(c):
        return msg_hbm.at[idx_v.at[pl.ds(c * G_ROWS, G_ROWS)]]

    pltpu.async_copy(gsrc(0), rows0, sem0)

    def body(c2, carry):
        i = 2 * c2
        pltpu.async_copy(gsrc(i + 1), rows1, sem1)
        pltpu.make_async_copy(gsrc(i), rows0, sem0).wait()

        @pl.when(c2 > 0)
        def _():
            pltpu.make_async_copy(out0, out_hbm.at[pl.ds(abase, CA)],
                                  osem0).wait()

        def atom0(a, cc):
            for j in range(NCOL):
                acc = rows0[a * MAX_NB, pl.ds(j * L, L)]
                for k in range(1, MAX_NB):
                    acc = acc + rows0[a * MAX_NB + k, pl.ds(j * L, L)]
                out0[a, pl.ds(j * L, L)] = acc
            return cc

        lax.fori_loop(0, CA, atom0, 0)
        pltpu.async_copy(out0, out_hbm.at[pl.ds(abase + i * CA, CA)], osem0)

        nxt = jnp.minimum(i + 2, A_CHUNKS - 1)
        pltpu.async_copy(gsrc(nxt), rows0, sem0)
        pltpu.make_async_copy(gsrc(i + 1), rows1, sem1).wait()

        @pl.when(c2 > 0)
        def _():
            pltpu.make_async_copy(out1, out_hbm.at[pl.ds(abase, CA)],
                                  osem1).wait()

        def atom1(a, cc):
            for j in range(NCOL):
                acc = rows1[a * MAX_NB, pl.ds(j * L, L)]
                for k in range(1, MAX_NB):
                    acc = acc + rows1[a * MAX_NB + k, pl.ds(j * L, L)]
                out1[a, pl.ds(j * L, L)] = acc
            return cc

        lax.fori_loop(0, CA, atom1, 0)
        pltpu.async_copy(out1, out_hbm.at[pl.ds(abase + (i + 1) * CA, CA)],
                         osem1)
        return carry

    lax.fori_loop(0, A_CHUNKS // 2, body, 0)
    pltpu.make_async_copy(gsrc(A_CHUNKS - 1), rows0, sem0).wait()
    pltpu.make_async_copy(out0, out_hbm.at[pl.ds(abase, CA)], osem0).wait()
    pltpu.make_async_copy(out1, out_hbm.at[pl.ds(abase, CA)], osem1).wait()


@functools.cache
def _gather_sum():
    return pl.kernel(
        _gather_sum_body,
        out_type=jax.ShapeDtypeStruct((A_PAD, H), jnp.float32),
        mesh=_sc_mesh(),
        scratch_types=[
            pltpu.VMEM((APW * MAX_NB,), jnp.int32),
            pltpu.VMEM((G_ROWS, H), jnp.float32),
            pltpu.VMEM((G_ROWS, H), jnp.float32),
            pltpu.VMEM((CA, H), jnp.float32),
            pltpu.VMEM((CA, H), jnp.float32),
            pltpu.SemaphoreType.DMA,
            pltpu.SemaphoreType.DMA,
            pltpu.SemaphoreType.DMA,
            pltpu.SemaphoreType.DMA,
        ],
    )


CB = 40
B_CHUNKS = 125
BPW = CB * B_CHUNKS


def _edge_update_body(amsg_hbm, msg_hbm, b2a_hbm, b2revb_hbm, out_hbm,
                      ia_v, ir_v, ra0, ra1, rr0, rr1, mv0, mv1,
                      sa0, sa1, sr0, sr1, os0, os1):
    wid = lax.axis_index("s") * NC + lax.axis_index("c")
    bbase = wid * BPW
    pltpu.sync_copy(b2a_hbm.at[pl.ds(bbase, BPW)], ia_v)
    pltpu.sync_copy(b2revb_hbm.at[pl.ds(bbase, BPW)], ir_v)

    def asrc(c):
        return amsg_hbm.at[ia_v.at[pl.ds(c * CB, CB)]]

    def rsrc(c):
        return msg_hbm.at[ir_v.at[pl.ds(c * CB, CB)]]

    def start(c, ra, rr, sa, sr):
        pltpu.async_copy(asrc(c), ra, sa)
        pltpu.async_copy(rsrc(c), rr, sr)

    def sub(c, ra, rr, mv, osem):
        def row(r, cc):
            for j in range(NCOL):
                mv[r, pl.ds(j * L, L)] = (ra[r, pl.ds(j * L, L)]
                                          - rr[r, pl.ds(j * L, L)])
            return cc

        lax.fori_loop(0, CB, row, 0)
        pltpu.async_copy(mv, out_hbm.at[pl.ds(bbase + c * CB, CB)], osem)

    start(0, ra0, rr0, sa0, sr0)

    def body(c2, carry):
        i = 2 * c2
        start(i + 1, ra1, rr1, sa1, sr1)
        pltpu.make_async_copy(asrc(i), ra0, sa0).wait()
        pltpu.make_async_copy(rsrc(i), rr0, sr0).wait()

        @pl.when(c2 > 0)
        def _():
            pltpu.make_async_copy(mv0, out_hbm.at[pl.ds(bbase, CB)],
                                  os0).wait()

        sub(i, ra0, rr0, mv0, os0)
        start(i + 2, ra0, rr0, sa0, sr0)
        pltpu.make_async_copy(asrc(i + 1), ra1, sa1).wait()
        pltpu.make_async_copy(rsrc(i + 1), rr1, sr1).wait()

        @pl.when(c2 > 0)
        def _():
            pltpu.make_async_copy(mv1, out_hbm.at[pl.ds(bbase, CB)],
                                  os1).wait()

        sub(i + 1, ra1, rr1, mv1, os1)
        return carry

    lax.fori_loop(0, (B_CHUNKS - 1) // 2, body, 0)
    pltpu.make_async_copy(asrc(B_CHUNKS - 1), ra0, sa0).wait()
    pltpu.make_async_copy(rsrc(B_CHUNKS - 1), rr0, sr0).wait()
    pltpu.make_async_copy(mv0, out_hbm.at[pl.ds(bbase, CB)], os0).wait()
    sub(B_CHUNKS - 1, ra0, rr0, mv0, os0)
    pltpu.make_async_copy(mv0, out_hbm.at[pl.ds(bbase, CB)], os0).wait()
    pltpu.make_async_copy(mv1, out_hbm.at[pl.ds(bbase, CB)], os1).wait()


@functools.cache
def _edge_update():
    return pl.kernel(
        _edge_update_body,
        out_type=jax.ShapeDtypeStruct((N_BONDS, H), jnp.float32),
        mesh=_sc_mesh(),
        scratch_types=[
            pltpu.VMEM((BPW,), jnp.int32),
            pltpu.VMEM((BPW,), jnp.int32),
            pltpu.VMEM((CB, H), jnp.float32),
            pltpu.VMEM((CB, H), jnp.float32),
            pltpu.VMEM((CB, H), jnp.float32),
            pltpu.VMEM((CB, H), jnp.float32),
            pltpu.VMEM((CB, H), jnp.float32),
            pltpu.VMEM((CB, H), jnp.float32),
            pltpu.SemaphoreType.DMA,
            pltpu.SemaphoreType.DMA,
            pltpu.SemaphoreType.DMA,
            pltpu.SemaphoreType.DMA,
            pltpu.SemaphoreType.DMA,
            pltpu.SemaphoreType.DMA,
        ],
    )


BB_A = 1600


def _bf(x):
    return x.astype(jnp.bfloat16)


def _proj_inp_body(fb_ref, wi_ref, inp_ref):
    inp_ref[...] = jnp.dot(_bf(fb_ref[...]), wi_ref[...],
                           preferred_element_type=jnp.float32)


def _proj_inp(f_bonds, w_i_t):
    fdim = f_bonds.shape[1]
    return pl.pallas_call(
        _proj_inp_body,
        grid=(N_BONDS // BB_A,),
        in_specs=[
            pl.BlockSpec((BB_A, fdim), lambda i: (i, 0)),
            pl.BlockSpec((fdim, H), lambda i: (0, 0)),
        ],
        out_specs=pl.BlockSpec((BB_A, H), lambda i: (i, 0)),
        out_shape=jax.ShapeDtypeStruct((N_BONDS, H), jnp.float32),
    )(f_bonds, w_i_t)


BB_D = 1600


def _gru_body(inp_ref, m_ref, wih_ref, whh_ref, bih_ref, bhh_ref, out_ref):
    m = m_ref[...]
    xb = _bf(inp_ref[...])
    mb = _bf(m)
    wih = wih_ref[...]
    whh = whh_ref[...]
    bih = bih_ref[...]
    bhh = bhh_ref[...]
    pre_r = (jnp.dot(xb, wih[:, :H], preferred_element_type=jnp.float32)
             + jnp.dot(mb, whh[:, :H], preferred_element_type=jnp.float32)
             + bih[:, :H] + bhh[:, :H])
    r = jax.nn.sigmoid(pre_r)
    pre_z = (jnp.dot(xb, wih[:, H:2 * H], preferred_element_type=jnp.float32)
             + jnp.dot(mb, whh[:, H:2 * H], preferred_element_type=jnp.float32)
             + bih[:, H:2 * H] + bhh[:, H:2 * H])
    z = jax.nn.sigmoid(pre_z)
    gh_n = (jnp.dot(mb, whh[:, 2 * H:], preferred_element_type=jnp.float32)
            + bhh[:, 2 * H:])
    gi_n = (jnp.dot(xb, wih[:, 2 * H:], preferred_element_type=jnp.float32)
            + bih[:, 2 * H:])
    n = jnp.tanh(gi_n + r * gh_n)
    out_ref[...] = (1.0 - z) * n + z * m

    @pl.when(pl.program_id(0) == 0)
    def _():
        out_ref[0:1, :] = jnp.zeros((1, H), jnp.float32)


def _gru(inp, m, w_ih_t, w_hh_t, b_ih_row, b_hh_row):
    return pl.pallas_call(
        _gru_body,
        grid=(N_BONDS // BB_D,),
        in_specs=[
            pl.BlockSpec((BB_D, H), lambda i: (i, 0)),
            pl.BlockSpec((BB_D, H), lambda i: (i, 0)),
            pl.BlockSpec((H, 3 * H), lambda i: (0, 0)),
            pl.BlockSpec((H, 3 * H), lambda i: (0, 0)),
            pl.BlockSpec((1, 3 * H), lambda i: (0, 0)),
            pl.BlockSpec((1, 3 * H), lambda i: (0, 0)),
        ],
        out_specs=pl.BlockSpec((BB_D, H), lambda i: (i, 0)),
        out_shape=jax.ShapeDtypeStruct((N_BONDS, H), jnp.float32),
    )(inp, m, w_ih_t, w_hh_t, b_ih_row, b_hh_row)


BA_E = 2000


def _out_body(fa_ref, am_ref, w1_ref, w2_ref, b_ref, mask_ref, out_ref):
    acc = jnp.dot(_bf(fa_ref[...]), w1_ref[...],
                  preferred_element_type=jnp.float32)
    acc = acc + jnp.dot(_bf(am_ref[...]), w2_ref[...],
                        preferred_element_type=jnp.float32)
    acc = jnp.maximum(acc + b_ref[...], 0.0)
    out_ref[...] = acc * mask_ref[...]


def _out_proj(f_atoms, a_msg_pad, w1_t, w2_t, b_row, mask):
    return pl.pallas_call(
        _out_body,
        grid=(N_ATOMS // BA_E,),
        in_specs=[
            pl.BlockSpec((BA_E, f_atoms.shape[1]), lambda i: (i, 0)),
            pl.BlockSpec((BA_E, H), lambda i: (i, 0)),
            pl.BlockSpec((f_atoms.shape[1], H), lambda i: (0, 0)),
            pl.BlockSpec((H, H), lambda i: (0, 0)),
            pl.BlockSpec((1, H), lambda i: (0, 0)),
            pl.BlockSpec((BA_E, 1), lambda i: (i, 0)),
        ],
        out_specs=pl.BlockSpec((BA_E, H), lambda i: (i, 0)),
        out_shape=jax.ShapeDtypeStruct((N_ATOMS, H), jnp.float32),
    )(f_atoms, a_msg_pad, w1_t, w2_t, b_row, mask)


def kernel(f_atoms, f_bonds, a2b, b2a, b2revb, undirected_b2a, mask,
           W_i, W_ih, W_hh, b_ih, b_hh, W_o_w, W_o_b):
    del undirected_b2a
    afdim = f_atoms.shape[1]
    w_i_t = _bf(W_i.T)
    w_ih_t = _bf(W_ih.T)
    w_hh_t = _bf(W_hh.T)
    w1_t = _bf(W_o_w[:, :afdim].T)
    w2_t = _bf(W_o_w[:, afdim:].T)

    n_pad = A_PAD * MAX_NB - N_ATOMS * MAX_NB
    a2b_flat = jnp.concatenate([
        a2b.reshape(-1).astype(jnp.int32),
        jnp.arange(n_pad, dtype=jnp.int32),
    ])
    b2a = b2a.astype(jnp.int32)
    b2revb = b2revb.astype(jnp.int32)

    inp = _proj_inp(f_bonds, w_i_t)

    msg = inp
    for _ in range(DEPTH - 1):
        amsg = _gather_sum()(msg, a2b_flat)
        m = _edge_update()(amsg, msg, b2a, b2revb)
        msg = _gru(inp, m, w_ih_t, w_hh_t,
                   b_ih.reshape(1, -1), b_hh.reshape(1, -1))

    amsg = _gather_sum()(msg, a2b_flat)
    return _out_proj(f_atoms, amsg, w1_t, w2_t, W_o_b.reshape(1, -1), mask)


# speedup vs baseline: 1.2416x; 1.0313x over previous
import functools

import jax
import jax.numpy as jnp
from jax import lax
from jax.experimental import pallas as pl
from jax.experimental.pallas import tpu as pltpu
from jax.experimental.pallas import tpu_sc as plsc

N_ATOMS = 10000
N_BONDS = 160000
MAX_NB = 16
H = 256
DEPTH = 3

NC, NS, L = 2, 16, 16
NW = NC * NS
NCOL = H // L


@functools.cache
def _sc_mesh():
    return plsc.VectorSubcoreMesh(core_axis_name="c", subcore_axis_name="s")


CA = 8
G_ROWS = CA * MAX_NB
A_CHUNKS = 40
APW = CA * A_CHUNKS
A_PAD = APW * NW


def _gather_sum_body(msg_hbm, a2b_hbm, out_hbm, idx_v,
                     rows0, rows1, out0, out1, sem0, sem1, osem0, osem1):
    wid = lax.axis_index("s") * NC + lax.axis_index("c")
    abase = wid * APW
    pltpu.sync_copy(a2b_hbm.at[pl.ds(abase * MAX_NB, APW * MAX_NB)], idx_v)

    def gsrc(c):
        return msg_hbm.at[idx_v.at[pl.ds(c * G_ROWS, G_ROWS)]]

    pltpu.async_copy(gsrc(0), rows0, sem0)

    def body(c2, carry):
        i = 2 * c2
        pltpu.async_copy(gsrc(i + 1), rows1, sem1)
        pltpu.make_async_copy(gsrc(i), rows0, sem0).wait()

        @pl.when(c2 > 0)
        def _():
            pltpu.make_async_copy(out0, out_hbm.at[pl.ds(abase, CA)],
                                  osem0).wait()

        def atom0(a, cc):
            for j in range(NCOL):
                acc = rows0[a * MAX_NB, pl.ds(j * L, L)]
                for k in range(1, MAX_NB):
                    acc = acc + rows0[a * MAX_NB + k, pl.ds(j * L, L)]
                out0[a, pl.ds(j * L, L)] = acc
            return cc

        lax.fori_loop(0, CA, atom0, 0)
        pltpu.async_copy(out0, out_hbm.at[pl.ds(abase + i * CA, CA)], osem0)

        nxt = jnp.minimum(i + 2, A_CHUNKS - 1)
        pltpu.async_copy(gsrc(nxt), rows0, sem0)
        pltpu.make_async_copy(gsrc(i + 1), rows1, sem1).wait()

        @pl.when(c2 > 0)
        def _():
            pltpu.make_async_copy(out1, out_hbm.at[pl.ds(abase, CA)],
                                  osem1).wait()

        def atom1(a, cc):
            for j in range(NCOL):
                acc = rows1[a * MAX_NB, pl.ds(j * L, L)]
                for k in range(1, MAX_NB):
                    acc = acc + rows1[a * MAX_NB + k, pl.ds(j * L, L)]
                out1[a, pl.ds(j * L, L)] = acc
            return cc

        lax.fori_loop(0, CA, atom1, 0)
        pltpu.async_copy(out1, out_hbm.at[pl.ds(abase + (i + 1) * CA, CA)],
                         osem1)
        return carry

    lax.fori_loop(0, A_CHUNKS // 2, body, 0)
    pltpu.make_async_copy(gsrc(A_CHUNKS - 1), rows0, sem0).wait()
    pltpu.make_async_copy(out0, out_hbm.at[pl.ds(abase, CA)], osem0).wait()
    pltpu.make_async_copy(out1, out_hbm.at[pl.ds(abase, CA)], osem1).wait()


@functools.cache
def _gather_sum():
    return pl.kernel(
        _gather_sum_body,
        out_type=jax.ShapeDtypeStruct((A_PAD, H), jnp.float32),
        mesh=_sc_mesh(),
        scratch_types=[
            pltpu.VMEM((APW * MAX_NB,), jnp.int32),
            pltpu.VMEM((G_ROWS, H), jnp.float32),
            pltpu.VMEM((G_ROWS, H), jnp.float32),
            pltpu.VMEM((CA, H), jnp.float32),
            pltpu.VMEM((CA, H), jnp.float32),
            pltpu.SemaphoreType.DMA,
            pltpu.SemaphoreType.DMA,
            pltpu.SemaphoreType.DMA,
            pltpu.SemaphoreType.DMA,
        ],
    )


CB = 40


def _make_edge_body(start, bpw, nch):
    def body(amsg_hbm, msg_hbm, b2a_hbm, b2revb_hbm, out_hbm,
             ia_v, ir_v, ra0, ra1, rr0, rr1, mv0, mv1,
             sa0, sa1, sr0, sr1, os0, os1):
        wid = lax.axis_index("s") * NC + lax.axis_index("c")
        ibase = start + wid * bpw
        obase = wid * bpw
        pltpu.sync_copy(b2a_hbm.at[pl.ds(ibase, bpw)], ia_v)
        pltpu.sync_copy(b2revb_hbm.at[pl.ds(ibase, bpw)], ir_v)

        def asrc(c):
            return amsg_hbm.at[ia_v.at[pl.ds(c * CB, CB)]]

        def rsrc(c):
            return msg_hbm.at[ir_v.at[pl.ds(c * CB, CB)]]

        def start_pair(c, ra, rr, sa, sr):
            pltpu.async_copy(asrc(c), ra, sa)
            pltpu.async_copy(rsrc(c), rr, sr)

        def sub(c, ra, rr, mv, osem):
            def row(r, cc):
                for j in range(NCOL):
                    mv[r, pl.ds(j * L, L)] = (ra[r, pl.ds(j * L, L)]
                                              - rr[r, pl.ds(j * L, L)])
                return cc

            lax.fori_loop(0, CB, row, 0)
            pltpu.async_copy(mv, out_hbm.at[pl.ds(obase + c * CB, CB)], osem)

        start_pair(0, ra0, rr0, sa0, sr0)

        def loop_body(c2, carry):
            i = 2 * c2
            start_pair(i + 1, ra1, rr1, sa1, sr1)
            pltpu.make_async_copy(asrc(i), ra0, sa0).wait()
            pltpu.make_async_copy(rsrc(i), rr0, sr0).wait()

            @pl.when(c2 > 0)
            def _():
                pltpu.make_async_copy(mv0, out_hbm.at[pl.ds(obase, CB)],
                                      os0).wait()

            sub(i, ra0, rr0, mv0, os0)
            start_pair(jnp.minimum(i + 2, nch - 1), ra0, rr0, sa0, sr0)
            pltpu.make_async_copy(asrc(i + 1), ra1, sa1).wait()
            pltpu.make_async_copy(rsrc(i + 1), rr1, sr1).wait()

            @pl.when(c2 > 0)
            def _():
                pltpu.make_async_copy(mv1, out_hbm.at[pl.ds(obase, CB)],
                                      os1).wait()

            sub(i + 1, ra1, rr1, mv1, os1)
            return carry

        lax.fori_loop(0, nch // 2, loop_body, 0)
        pltpu.make_async_copy(asrc(nch - 1), ra0, sa0).wait()
        pltpu.make_async_copy(rsrc(nch - 1), rr0, sr0).wait()
        if nch % 2:
            pltpu.make_async_copy(mv0, out_hbm.at[pl.ds(obase, CB)],
                                  os0).wait()
            sub(nch - 1, ra0, rr0, mv0, os0)
        pltpu.make_async_copy(mv0, out_hbm.at[pl.ds(obase, CB)], os0).wait()
        pltpu.make_async_copy(mv1, out_hbm.at[pl.ds(obase, CB)], os1).wait()

    return body


@functools.cache
def _edge_update(start, n):
    bpw = n // NW
    nch = bpw // CB
    return pl.kernel(
        _make_edge_body(start, bpw, nch),
        out_type=jax.ShapeDtypeStruct((n, H), jnp.float32),
        mesh=_sc_mesh(),
        scratch_types=[
            pltpu.VMEM((bpw,), jnp.int32),
            pltpu.VMEM((bpw,), jnp.int32),
            pltpu.VMEM((CB, H), jnp.float32),
            pltpu.VMEM((CB, H), jnp.float32),
            pltpu.VMEM((CB, H), jnp.float32),
            pltpu.VMEM((CB, H), jnp.float32),
            pltpu.VMEM((CB, H), jnp.float32),
            pltpu.VMEM((CB, H), jnp.float32),
            pltpu.SemaphoreType.DMA,
            pltpu.SemaphoreType.DMA,
            pltpu.SemaphoreType.DMA,
            pltpu.SemaphoreType.DMA,
            pltpu.SemaphoreType.DMA,
            pltpu.SemaphoreType.DMA,
        ],
    )


BB_A = 1600


def _bf(x):
    return x.astype(jnp.bfloat16)


def _proj_inp_body(fb_ref, wi_ref, inp_ref):
    inp_ref[...] = jnp.dot(_bf(fb_ref[...]), wi_ref[...],
                           preferred_element_type=jnp.float32)


def _proj_inp(f_bonds, w_i_t):
    fdim = f_bonds.shape[1]
    return pl.pallas_call(
        _proj_inp_body,
        grid=(N_BONDS // BB_A,),
        in_specs=[
            pl.BlockSpec((BB_A, fdim), lambda i: (i, 0)),
            pl.BlockSpec((fdim, H), lambda i: (0, 0)),
        ],
        out_specs=pl.BlockSpec((BB_A, H), lambda i: (i, 0)),
        out_shape=jax.ShapeDtypeStruct((N_BONDS, H), jnp.float32),
    )(f_bonds, w_i_t)


BB_D = 1280
B_SPLIT = 81920


def _gru_block(inp_ref, m_ref, wih_ref, whh_ref, bih_ref, bhh_ref, out_ref,
               zero_row0):
    m = m_ref[...]
    gi = (jnp.dot(_bf(inp_ref[...]), wih_ref[...],
                  preferred_element_type=jnp.float32) + bih_ref[...])
    gh = (jnp.dot(_bf(m), whh_ref[...], preferred_element_type=jnp.float32)
          + bhh_ref[...])
    r = jax.nn.sigmoid(gi[:, :H] + gh[:, :H])
    z = jax.nn.sigmoid(gi[:, H:2 * H] + gh[:, H:2 * H])
    n = jnp.tanh(gi[:, 2 * H:] + r * gh[:, 2 * H:])
    out_ref[...] = (1.0 - z) * n + z * m

    if zero_row0:
        @pl.when(pl.program_id(0) == 0)
        def _():
            out_ref[0:1, :] = jnp.zeros((1, H), jnp.float32)


def _gru_body_a(inp_ref, m_ref, wih_ref, whh_ref, bih_ref, bhh_ref, out_ref):
    _gru_block(inp_ref, m_ref, wih_ref, whh_ref, bih_ref, bhh_ref, out_ref,
               zero_row0=True)


def _gru_body_b(prev_ref, inp_ref, m_ref, wih_ref, whh_ref, bih_ref, bhh_ref,
                out_ref):
    del prev_ref
    _gru_block(inp_ref, m_ref, wih_ref, whh_ref, bih_ref, bhh_ref, out_ref,
               zero_row0=False)


def _gru_a(inp, m_part, weights):
    nblk = B_SPLIT // BB_D
    return pl.pallas_call(
        _gru_body_a,
        grid=(nblk,),
        in_specs=[
            pl.BlockSpec((BB_D, H), lambda i: (i, 0)),
            pl.BlockSpec((BB_D, H), lambda i: (i, 0)),
            pl.BlockSpec((H, 3 * H), lambda i: (0, 0)),
            pl.BlockSpec((H, 3 * H), lambda i: (0, 0)),
            pl.BlockSpec((1, 3 * H), lambda i: (0, 0)),
            pl.BlockSpec((1, 3 * H), lambda i: (0, 0)),
        ],
        out_specs=pl.BlockSpec((BB_D, H), lambda i: (i, 0)),
        out_shape=jax.ShapeDtypeStruct((N_BONDS, H), jnp.float32),
    )(inp, m_part, *weights)


def _gru_b(prev, inp, m_part, weights):
    nblk = (N_BONDS - B_SPLIT) // BB_D
    off = B_SPLIT // BB_D
    return pl.pallas_call(
        _gru_body_b,
        grid=(nblk,),
        in_specs=[
            pl.BlockSpec((8, H), lambda i: (0, 0)),
            pl.BlockSpec((BB_D, H), lambda i: (i + off, 0)),
            pl.BlockSpec((BB_D, H), lambda i: (i, 0)),
            pl.BlockSpec((H, 3 * H), lambda i: (0, 0)),
            pl.BlockSpec((H, 3 * H), lambda i: (0, 0)),
            pl.BlockSpec((1, 3 * H), lambda i: (0, 0)),
            pl.BlockSpec((1, 3 * H), lambda i: (0, 0)),
        ],
        out_specs=pl.BlockSpec((BB_D, H), lambda i: (i + off, 0)),
        out_shape=jax.ShapeDtypeStruct((N_BONDS, H), jnp.float32),
        input_output_aliases={0: 0},
    )(prev, inp, m_part, *weights)


BA_E = 2000


def _out_body(fa_ref, am_ref, w1_ref, w2_ref, b_ref, mask_ref, out_ref):
    acc = jnp.dot(_bf(fa_ref[...]), w1_ref[...],
                  preferred_element_type=jnp.float32)
    acc = acc + jnp.dot(_bf(am_ref[...]), w2_ref[...],
                        preferred_element_type=jnp.float32)
    acc = jnp.maximum(acc + b_ref[...], 0.0)
    out_ref[...] = acc * mask_ref[...]


def _out_proj(f_atoms, a_msg_pad, w1_t, w2_t, b_row, mask):
    return pl.pallas_call(
        _out_body,
        grid=(N_ATOMS // BA_E,),
        in_specs=[
            pl.BlockSpec((BA_E, f_atoms.shape[1]), lambda i: (i, 0)),
            pl.BlockSpec((BA_E, H), lambda i: (i, 0)),
            pl.BlockSpec((f_atoms.shape[1], H), lambda i: (0, 0)),
            pl.BlockSpec((H, H), lambda i: (0, 0)),
            pl.BlockSpec((1, H), lambda i: (0, 0)),
            pl.BlockSpec((BA_E, 1), lambda i: (i, 0)),
        ],
        out_specs=pl.BlockSpec((BA_E, H), lambda i: (i, 0)),
        out_shape=jax.ShapeDtypeStruct((N_ATOMS, H), jnp.float32),
    )(f_atoms, a_msg_pad, w1_t, w2_t, b_row, mask)


def kernel(f_atoms, f_bonds, a2b, b2a, b2revb, undirected_b2a, mask,
           W_i, W_ih, W_hh, b_ih, b_hh, W_o_w, W_o_b):
    del undirected_b2a
    afdim = f_atoms.shape[1]
    w_i_t = _bf(W_i.T)
    w_ih_t = _bf(W_ih.T)
    w_hh_t = _bf(W_hh.T)
    w1_t = _bf(W_o_w[:, :afdim].T)
    w2_t = _bf(W_o_w[:, afdim:].T)

    n_pad = A_PAD * MAX_NB - N_ATOMS * MAX_NB
    a2b_flat = jnp.concatenate([
        a2b.reshape(-1).astype(jnp.int32),
        jnp.arange(n_pad, dtype=jnp.int32),
    ])
    b2a = b2a.astype(jnp.int32)
    b2revb = b2revb.astype(jnp.int32)

    gw = (w_ih_t, w_hh_t, b_ih.reshape(1, -1), b_hh.reshape(1, -1))
    inp = _proj_inp(f_bonds, w_i_t)

    msg = inp
    for _ in range(DEPTH - 1):
        amsg = _gather_sum()(msg, a2b_flat)
        m0 = _edge_update(0, B_SPLIT)(amsg, msg, b2a, b2revb)
        m1 = _edge_update(B_SPLIT, N_BONDS - B_SPLIT)(amsg, msg, b2a, b2revb)
        p0 = _gru_a(inp, m0, gw)
        msg = _gru_b(p0, inp, m1, gw)

    amsg = _gather_sum()(msg, a2b_flat)
    return _out_proj(f_atoms, amsg, w1_t, w2_t, W_o_b.reshape(1, -1), mask)


# speedup vs baseline: 1.2647x; 1.0187x over previous
import functools

import jax
import jax.numpy as jnp
from jax import lax
from jax.experimental import pallas as pl
from jax.experimental.pallas import tpu as pltpu
from jax.experimental.pallas import tpu_sc as plsc

N_ATOMS = 10000
N_BONDS = 160000
MAX_NB = 16
H = 256
DEPTH = 3

NC, NS, L = 2, 16, 16
NW = NC * NS
NCOL = H // L


@functools.cache
def _sc_mesh():
    return plsc.VectorSubcoreMesh(core_axis_name="c", subcore_axis_name="s")


CA = 8
G_ROWS = CA * MAX_NB
A_CHUNKS = 40
APW = CA * A_CHUNKS
A_PAD = APW * NW


def _gather_sum_body(msg_hbm, a2b_hbm, out_hbm, idx_v,
                     rows0, rows1, out0, out1, sem0, sem1, osem0, osem1):
    wid = lax.axis_index("s") * NC + lax.axis_index("c")
    abase = wid * APW
    pltpu.sync_copy(a2b_hbm.at[pl.ds(abase * MAX_NB, APW * MAX_NB)], idx_v)

    def gsrc(c):
        return msg_hbm.at[idx_v.at[pl.ds(c * G_ROWS, G_ROWS)]]

    pltpu.async_copy(gsrc(0), rows0, sem0)

    def body(c2, carry):
        i = 2 * c2
        pltpu.async_copy(gsrc(i + 1), rows1, sem1)
        pltpu.make_async_copy(gsrc(i), rows0, sem0).wait()

        @pl.when(c2 > 0)
        def _():
            pltpu.make_async_copy(out0, out_hbm.at[pl.ds(abase, CA)],
                                  osem0).wait()

        def atom0(a, cc):
            for j in range(NCOL):
                acc = rows0[a * MAX_NB, pl.ds(j * L, L)]
                for k in range(1, MAX_NB):
                    acc = acc + rows0[a * MAX_NB + k, pl.ds(j * L, L)]
                out0[a, pl.ds(j * L, L)] = acc
            return cc

        lax.fori_loop(0, CA, atom0, 0)
        pltpu.async_copy(out0, out_hbm.at[pl.ds(abase + i * CA, CA)], osem0)

        nxt = jnp.minimum(i + 2, A_CHUNKS - 1)
        pltpu.async_copy(gsrc(nxt), rows0, sem0)
        pltpu.make_async_copy(gsrc(i + 1), rows1, sem1).wait()

        @pl.when(c2 > 0)
        def _():
            pltpu.make_async_copy(out1, out_hbm.at[pl.ds(abase, CA)],
                                  osem1).wait()

        def atom1(a, cc):
            for j in range(NCOL):
                acc = rows1[a * MAX_NB, pl.ds(j * L, L)]
                for k in range(1, MAX_NB):
                    acc = acc + rows1[a * MAX_NB + k, pl.ds(j * L, L)]
                out1[a, pl.ds(j * L, L)] = acc
            return cc

        lax.fori_loop(0, CA, atom1, 0)
        pltpu.async_copy(out1, out_hbm.at[pl.ds(abase + (i + 1) * CA, CA)],
                         osem1)
        return carry

    lax.fori_loop(0, A_CHUNKS // 2, body, 0)
    pltpu.make_async_copy(gsrc(A_CHUNKS - 1), rows0, sem0).wait()
    pltpu.make_async_copy(out0, out_hbm.at[pl.ds(abase, CA)], osem0).wait()
    pltpu.make_async_copy(out1, out_hbm.at[pl.ds(abase, CA)], osem1).wait()


@functools.cache
def _gather_sum():
    return pl.kernel(
        _gather_sum_body,
        out_type=jax.ShapeDtypeStruct((A_PAD, H), jnp.float32),
        mesh=_sc_mesh(),
        scratch_types=[
            pltpu.VMEM((APW * MAX_NB,), jnp.int32),
            pltpu.VMEM((G_ROWS, H), jnp.float32),
            pltpu.VMEM((G_ROWS, H), jnp.float32),
            pltpu.VMEM((CA, H), jnp.float32),
            pltpu.VMEM((CA, H), jnp.float32),
            pltpu.SemaphoreType.DMA,
            pltpu.SemaphoreType.DMA,
            pltpu.SemaphoreType.DMA,
            pltpu.SemaphoreType.DMA,
        ],
    )


CB = 40


def _make_edge_body(start, bpw, nch):
    def body(amsg_hbm, msg_hbm, b2a_hbm, b2revb_hbm, out_hbm,
             ia_v, ir_v, ra0, ra1, rr0, rr1, mv0, mv1,
             sa0, sa1, sr0, sr1, os0, os1):
        wid = lax.axis_index("s") * NC + lax.axis_index("c")
        ibase = start + wid * bpw
        obase = wid * bpw
        pltpu.sync_copy(b2a_hbm.at[pl.ds(ibase, bpw)], ia_v)
        pltpu.sync_copy(b2revb_hbm.at[pl.ds(ibase, bpw)], ir_v)

        def asrc(c):
            return amsg_hbm.at[ia_v.at[pl.ds(c * CB, CB)]]

        def rsrc(c):
            return msg_hbm.at[ir_v.at[pl.ds(c * CB, CB)]]

        def start_pair(c, ra, rr, sa, sr):
            pltpu.async_copy(asrc(c), ra, sa)
            pltpu.async_copy(rsrc(c), rr, sr)

        def sub(c, ra, rr, mv, osem):
            def row(r, cc):
                for j in range(NCOL):
                    mv[r, pl.ds(j * L, L)] = (ra[r, pl.ds(j * L, L)]
                                              - rr[r, pl.ds(j * L, L)])
                return cc

            lax.fori_loop(0, CB, row, 0)
            pltpu.async_copy(mv, out_hbm.at[pl.ds(obase + c * CB, CB)], osem)

        start_pair(0, ra0, rr0, sa0, sr0)

        def loop_body(c2, carry):
            i = 2 * c2
            start_pair(i + 1, ra1, rr1, sa1, sr1)
            pltpu.make_async_copy(asrc(i), ra0, sa0).wait()
            pltpu.make_async_copy(rsrc(i), rr0, sr0).wait()

            @pl.when(c2 > 0)
            def _():
                pltpu.make_async_copy(mv0, out_hbm.at[pl.ds(obase, CB)],
                                      os0).wait()

            sub(i, ra0, rr0, mv0, os0)
            start_pair(jnp.minimum(i + 2, nch - 1), ra0, rr0, sa0, sr0)
            pltpu.make_async_copy(asrc(i + 1), ra1, sa1).wait()
            pltpu.make_async_copy(rsrc(i + 1), rr1, sr1).wait()

            @pl.when(c2 > 0)
            def _():
                pltpu.make_async_copy(mv1, out_hbm.at[pl.ds(obase, CB)],
                                      os1).wait()

            sub(i + 1, ra1, rr1, mv1, os1)
            return carry

        lax.fori_loop(0, nch // 2, loop_body, 0)
        pltpu.make_async_copy(asrc(nch - 1), ra0, sa0).wait()
        pltpu.make_async_copy(rsrc(nch - 1), rr0, sr0).wait()
        if nch % 2:
            pltpu.make_async_copy(mv0, out_hbm.at[pl.ds(obase, CB)],
                                  os0).wait()
            sub(nch - 1, ra0, rr0, mv0, os0)
        pltpu.make_async_copy(mv0, out_hbm.at[pl.ds(obase, CB)], os0).wait()
        pltpu.make_async_copy(mv1, out_hbm.at[pl.ds(obase, CB)], os1).wait()

    return body


@functools.cache
def _edge_update(start, n):
    bpw = n // NW
    nch = bpw // CB
    return pl.kernel(
        _make_edge_body(start, bpw, nch),
        out_type=jax.ShapeDtypeStruct((n, H), jnp.float32),
        mesh=_sc_mesh(),
        scratch_types=[
            pltpu.VMEM((bpw,), jnp.int32),
            pltpu.VMEM((bpw,), jnp.int32),
            pltpu.VMEM((CB, H), jnp.float32),
            pltpu.VMEM((CB, H), jnp.float32),
            pltpu.VMEM((CB, H), jnp.float32),
            pltpu.VMEM((CB, H), jnp.float32),
            pltpu.VMEM((CB, H), jnp.float32),
            pltpu.VMEM((CB, H), jnp.float32),
            pltpu.SemaphoreType.DMA,
            pltpu.SemaphoreType.DMA,
            pltpu.SemaphoreType.DMA,
            pltpu.SemaphoreType.DMA,
            pltpu.SemaphoreType.DMA,
            pltpu.SemaphoreType.DMA,
        ],
    )


BB_A = 3200


def _bf(x):
    return x.astype(jnp.bfloat16)


def _proj_inp_body(fb_ref, wi_ref, inp_ref):
    inp_ref[...] = jnp.dot(_bf(fb_ref[...]), wi_ref[...],
                           preferred_element_type=jnp.float32)


def _proj_inp(f_bonds, w_i_t):
    fdim = f_bonds.shape[1]
    return pl.pallas_call(
        _proj_inp_body,
        grid=(N_BONDS // BB_A,),
        in_specs=[
            pl.BlockSpec((BB_A, fdim), lambda i: (i, 0)),
            pl.BlockSpec((fdim, H), lambda i: (0, 0)),
        ],
        out_specs=pl.BlockSpec((BB_A, H), lambda i: (i, 0)),
        out_shape=jax.ShapeDtypeStruct((N_BONDS, H), jnp.float32),
    )(f_bonds, w_i_t)


BB_D = 1280
B_SPLIT = 81920


def _gru_block(inp_ref, m_ref, wih_ref, whh_ref, bih_ref, bhh_ref, out_ref,
               zero_row0):
    m = m_ref[...]
    gi = (jnp.dot(_bf(inp_ref[...]), wih_ref[...],
                  preferred_element_type=jnp.float32) + bih_ref[...])
    gh = (jnp.dot(_bf(m), whh_ref[...], preferred_element_type=jnp.float32)
          + bhh_ref[...])
    r = jax.nn.sigmoid(gi[:, :H] + gh[:, :H])
    z = jax.nn.sigmoid(gi[:, H:2 * H] + gh[:, H:2 * H])
    n = jnp.tanh(gi[:, 2 * H:] + r * gh[:, 2 * H:])
    out_ref[...] = (1.0 - z) * n + z * m

    if zero_row0:
        @pl.when(pl.program_id(0) == 0)
        def _():
            out_ref[0:1, :] = jnp.zeros((1, H), jnp.float32)


def _gru_body_a(inp_ref, m_ref, wih_ref, whh_ref, bih_ref, bhh_ref, out_ref):
    _gru_block(inp_ref, m_ref, wih_ref, whh_ref, bih_ref, bhh_ref, out_ref,
               zero_row0=True)


def _gru_body_b(prev_ref, inp_ref, m_ref, wih_ref, whh_ref, bih_ref, bhh_ref,
                out_ref):
    del prev_ref
    _gru_block(inp_ref, m_ref, wih_ref, whh_ref, bih_ref, bhh_ref, out_ref,
               zero_row0=False)


def _gru_a(inp, m_part, weights):
    nblk = B_SPLIT // BB_D
    return pl.pallas_call(
        _gru_body_a,
        grid=(nblk,),
        in_specs=[
            pl.BlockSpec((BB_D, H), lambda i: (i, 0)),
            pl.BlockSpec((BB_D, H), lambda i: (i, 0)),
            pl.BlockSpec((H, 3 * H), lambda i: (0, 0)),
            pl.BlockSpec((H, 3 * H), lambda i: (0, 0)),
            pl.BlockSpec((1, 3 * H), lambda i: (0, 0)),
            pl.BlockSpec((1, 3 * H), lambda i: (0, 0)),
        ],
        out_specs=pl.BlockSpec((BB_D, H), lambda i: (i, 0)),
        out_shape=jax.ShapeDtypeStruct((N_BONDS, H), jnp.float32),
    )(inp, m_part, *weights)


def _gru_b(prev, inp, m_part, weights):
    nblk = (N_BONDS - B_SPLIT) // BB_D
    off = B_SPLIT // BB_D
    return pl.pallas_call(
        _gru_body_b,
        grid=(nblk,),
        in_specs=[
            pl.BlockSpec((8, H), lambda i: (0, 0)),
            pl.BlockSpec((BB_D, H), lambda i: (i + off, 0)),
            pl.BlockSpec((BB_D, H), lambda i: (i, 0)),
            pl.BlockSpec((H, 3 * H), lambda i: (0, 0)),
            pl.BlockSpec((H, 3 * H), lambda i: (0, 0)),
            pl.BlockSpec((1, 3 * H), lambda i: (0, 0)),
            pl.BlockSpec((1, 3 * H), lambda i: (0, 0)),
        ],
        out_specs=pl.BlockSpec((BB_D, H), lambda i: (i + off, 0)),
        out_shape=jax.ShapeDtypeStruct((N_BONDS, H), jnp.float32),
        input_output_aliases={0: 0},
    )(prev, inp, m_part, *weights)


BA_E = 2000


def _out_body(fa_ref, am_ref, w1_ref, w2_ref, b_ref, mask_ref, out_ref):
    acc = jnp.dot(_bf(fa_ref[...]), w1_ref[...],
                  preferred_element_type=jnp.float32)
    acc = acc + jnp.dot(_bf(am_ref[...]), w2_ref[...],
                        preferred_element_type=jnp.float32)
    acc = jnp.maximum(acc + b_ref[...], 0.0)
    out_ref[...] = acc * mask_ref[...]


def _out_proj(f_atoms, a_msg_pad, w1_t, w2_t, b_row, mask):
    return pl.pallas_call(
        _out_body,
        grid=(N_ATOMS // BA_E,),
        in_specs=[
            pl.BlockSpec((BA_E, f_atoms.shape[1]), lambda i: (i, 0)),
            pl.BlockSpec((BA_E, H), lambda i: (i, 0)),
            pl.BlockSpec((f_atoms.shape[1], H), lambda i: (0, 0)),
            pl.BlockSpec((H, H), lambda i: (0, 0)),
            pl.BlockSpec((1, H), lambda i: (0, 0)),
            pl.BlockSpec((BA_E, 1), lambda i: (i, 0)),
        ],
        out_specs=pl.BlockSpec((BA_E, H), lambda i: (i, 0)),
        out_shape=jax.ShapeDtypeStruct((N_ATOMS, H), jnp.float32),
    )(f_atoms, a_msg_pad, w1_t, w2_t, b_row, mask)


def kernel(f_atoms, f_bonds, a2b, b2a, b2revb, undirected_b2a, mask,
           W_i, W_ih, W_hh, b_ih, b_hh, W_o_w, W_o_b):
    del undirected_b2a
    afdim = f_atoms.shape[1]
    w_i_t = _bf(W_i.T)
    w_ih_t = _bf(W_ih.T)
    w_hh_t = _bf(W_hh.T)
    w1_t = _bf(W_o_w[:, :afdim].T)
    w2_t = _bf(W_o_w[:, afdim:].T)

    n_pad = A_PAD * MAX_NB - N_ATOMS * MAX_NB
    a2b_flat = jnp.concatenate([
        a2b.reshape(-1).astype(jnp.int32),
        jnp.arange(n_pad, dtype=jnp.int32),
    ])
    b2a = b2a.astype(jnp.int32)
    b2revb = b2revb.astype(jnp.int32)

    gw = (w_ih_t, w_hh_t, b_ih.reshape(1, -1), b_hh.reshape(1, -1))
    inp = _proj_inp(f_bonds, w_i_t)

    msg = inp
    for _ in range(DEPTH - 1):
        amsg = _gather_sum()(msg, a2b_flat)
        m0 = _edge_update(0, B_SPLIT)(amsg, msg, b2a, b2revb)
        m1 = _edge_update(B_SPLIT, N_BONDS - B_SPLIT)(amsg, msg, b2a, b2revb)
        p0 = _gru_a(inp, m0, gw)
        msg = _gru_b(p0, inp, m1, gw)

    amsg = _gather_sum()(msg, a2b_flat)
    return _out_proj(f_atoms, amsg, w1_t, w2_t, W_o_b.reshape(1, -1), mask)
